# Initial kernel scaffold; baseline (speedup 1.0000x reference)
import jax
import jax.numpy as jnp
from jax.experimental import pallas as pl


def kernel(x, target):
    raise NotImplementedError("write your pallas kernel here")


# speedup vs baseline: 2.3269x; 2.3269x over previous
import math

import jax
import jax.numpy as jnp
from jax import lax
from jax.experimental import pallas as pl
from jax.experimental.pallas import tpu as pltpu
from jax.experimental.pallas import tpu_sc as plsc

N_TOK = 4096
N_CLS = 32000
PAD = 0
SMOOTHING = 0.1
CONF = 1.0 - SMOOTHING
EPS = SMOOTHING / (N_CLS - 2)
C0 = (N_CLS - 2) * EPS * math.log(EPS) + CONF * math.log(CONF)

ROW_BLK = 512
COL_BLK = 3200


def _masked_sum_body(tgt_ref, x_ref, acc_ref):
    @pl.when((pl.program_id(0) == 0) & (pl.program_id(1) == 0))
    def _():
        acc_ref[0, 0] = 0.0

    m = (tgt_ref[...] != PAD).astype(jnp.float32)
    acc_ref[0, 0] += jnp.sum(x_ref[...] * m)


_masked_sum = pl.pallas_call(
    _masked_sum_body,
    grid=(N_TOK // ROW_BLK, N_CLS // COL_BLK),
    in_specs=[
        pl.BlockSpec((ROW_BLK, 1), lambda i, j: (i, 0)),
        pl.BlockSpec((ROW_BLK, COL_BLK), lambda i, j: (i, j)),
    ],
    out_specs=pl.BlockSpec((1, 1), lambda i, j: (0, 0), memory_space=pltpu.SMEM),
    out_shape=jax.ShapeDtypeStruct((1, 1), jnp.float32),
)

L = 16
NC, NS = 2, 16
NW = NC * NS
BPW = N_TOK // NW
RPS = N_CLS // L


def _sc_gather_body(x_hbm, tgt_hbm, out_hbm, tgt_v, idx_v, idx0_v,
                    vals_v, vals0_v, res_v, sem):
    wid = lax.axis_index("s") * NC + lax.axis_index("c")
    base = wid * BPW
    pltpu.sync_copy(tgt_hbm.at[pl.ds(base, BPW)], tgt_v)

    lane_ids = lax.iota(jnp.int32, L)
    for j in range(BPW // L):
        t = tgt_v[pl.ds(j * L, L)]
        row_start = (base + j * L + lane_ids) * N_CLS
        idx_v[pl.ds(j * L, L)] = row_start + t
        idx0_v[pl.ds(j * L, L)] = row_start

    pltpu.async_copy(x_hbm.at[idx_v], vals_v, sem).wait()
    pltpu.async_copy(x_hbm.at[idx0_v], vals0_v, sem).wait()

    zero = jnp.zeros((L,), jnp.float32)
    accg = zero
    accb = zero
    acck = zero
    for j in range(BPW // L):
        sl = pl.ds(j * L, L)
        valid = tgt_v[sl] != PAD
        accg = accg + jnp.where(valid, vals_v[sl], 0.0)
        accb = accb + jnp.where(valid, vals0_v[sl], 0.0)
        acck = acck + jnp.where(valid, 1.0, 0.0)

    res_v[0, :] = accg
    res_v[1, :] = accb
    res_v[2, :] = acck
    pltpu.sync_copy(res_v, out_hbm.at[wid])


_sc_gather = pl.kernel(
    _sc_gather_body,
    out_type=jax.ShapeDtypeStruct((NW, 3, L), jnp.float32),
    mesh=plsc.VectorSubcoreMesh(core_axis_name="c", subcore_axis_name="s"),
    scratch_types=[
        pltpu.VMEM((BPW,), jnp.int32),
        pltpu.VMEM((BPW,), jnp.int32),
        pltpu.VMEM((BPW,), jnp.int32),
        pltpu.VMEM((BPW,), jnp.float32),
        pltpu.VMEM((BPW,), jnp.float32),
        pltpu.VMEM((3, L), jnp.float32),
        pltpu.SemaphoreType.DMA,
    ],
)


def kernel(x, target):
    tgt = target.astype(jnp.int32)
    a = _masked_sum(tgt.reshape(N_TOK, 1), x)[0, 0]
    res = _sc_gather(x.reshape(N_TOK * N_CLS), tgt)
    g = jnp.sum(res[:, 0, :])
    b = jnp.sum(res[:, 1, :])
    k = jnp.sum(res[:, 2, :])
    return k * C0 - EPS * a + EPS * b + (EPS - CONF) * g


# speedup vs baseline: 2.4689x; 1.0610x over previous
import math

import jax
import jax.numpy as jnp
from jax import lax
from jax.experimental import pallas as pl
from jax.experimental.pallas import tpu as pltpu
from jax.experimental.pallas import tpu_sc as plsc

N_TOK = 4096
N_CLS = 32000
PAD = 0
SMOOTHING = 0.1
CONF = 1.0 - SMOOTHING
EPS = SMOOTHING / (N_CLS - 2)
C0 = (N_CLS - 2) * EPS * math.log(EPS) + CONF * math.log(CONF)

ROW_BLK = 128
COL_BLK = 32000


def _masked_sum_body(tgt_ref, x_ref, acc_ref):
    @pl.when(pl.program_id(0) == 0)
    def _():
        acc_ref[0, 0] = 0.0

    m = (tgt_ref[...] != PAD).astype(jnp.float32)
    acc_ref[0, 0] += jnp.sum(x_ref[...] * m)


_masked_sum = pl.pallas_call(
    _masked_sum_body,
    grid=(N_TOK // ROW_BLK,),
    in_specs=[
        pl.BlockSpec((ROW_BLK, 1), lambda i: (i, 0)),
        pl.BlockSpec((ROW_BLK, COL_BLK), lambda i: (i, 0)),
    ],
    out_specs=pl.BlockSpec((1, 1), lambda i: (0, 0), memory_space=pltpu.SMEM),
    out_shape=jax.ShapeDtypeStruct((1, 1), jnp.float32),
)

L = 16
NC, NS = 2, 16
NW = NC * NS
BPW = N_TOK // NW
RPS = N_CLS // L


def _sc_gather_body(x_hbm, tgt_hbm, out_hbm, tgt_v, idx_v, idx0_v,
                    vals_v, vals0_v, res_v, sem):
    wid = lax.axis_index("s") * NC + lax.axis_index("c")
    base = wid * BPW
    pltpu.sync_copy(tgt_hbm.at[pl.ds(base, BPW)], tgt_v)

    lane_ids = lax.iota(jnp.int32, L)
    for j in range(BPW // L):
        t = tgt_v[pl.ds(j * L, L)]
        row_start = (base + j * L + lane_ids) * N_CLS
        idx_v[pl.ds(j * L, L)] = row_start + t
        idx0_v[pl.ds(j * L, L)] = row_start

    pltpu.async_copy(x_hbm.at[idx_v], vals_v, sem).wait()
    pltpu.async_copy(x_hbm.at[idx0_v], vals0_v, sem).wait()

    zero = jnp.zeros((L,), jnp.float32)
    accg = zero
    accb = zero
    acck = zero
    for j in range(BPW // L):
        sl = pl.ds(j * L, L)
        valid = tgt_v[sl] != PAD
        accg = accg + jnp.where(valid, vals_v[sl], 0.0)
        accb = accb + jnp.where(valid, vals0_v[sl], 0.0)
        acck = acck + jnp.where(valid, 1.0, 0.0)

    res_v[0, :] = accg
    res_v[1, :] = accb
    res_v[2, :] = acck
    pltpu.sync_copy(res_v, out_hbm.at[wid])


_sc_gather = pl.kernel(
    _sc_gather_body,
    out_type=jax.ShapeDtypeStruct((NW, 3, L), jnp.float32),
    mesh=plsc.VectorSubcoreMesh(core_axis_name="c", subcore_axis_name="s"),
    scratch_types=[
        pltpu.VMEM((BPW,), jnp.int32),
        pltpu.VMEM((BPW,), jnp.int32),
        pltpu.VMEM((BPW,), jnp.int32),
        pltpu.VMEM((BPW,), jnp.float32),
        pltpu.VMEM((BPW,), jnp.float32),
        pltpu.VMEM((3, L), jnp.float32),
        pltpu.SemaphoreType.DMA,
    ],
)


def kernel(x, target):
    tgt = target.astype(jnp.int32)
    a = _masked_sum(tgt.reshape(N_TOK, 1), x)[0, 0]
    res = _sc_gather(x.reshape(N_TOK * N_CLS), tgt)
    g = jnp.sum(res[:, 0, :])
    b = jnp.sum(res[:, 1, :])
    k = jnp.sum(res[:, 2, :])
    return k * C0 - EPS * a + EPS * b + (EPS - CONF) * g
